# Initial kernel scaffold; baseline (speedup 1.0000x reference)
import jax
import jax.numpy as jnp
from jax.experimental import pallas as pl


def kernel(he_feat, maccs_feat, pubchem_feat, erg_feat, src_maccs, dst_maccs, src_pubchem, dst_pubchem, src_erg, dst_erg, W1, b1, W2, b2, W3, b3, W4, b4, W5m, b5m, W5p, b5p, W5e, b5e, W6m, b6m, W7m, b7m, W6p, b6p, W7p, b7p, W6e, b6e, W7e, b7e, Wm1, bm1, Wm2, bm2):
    raise NotImplementedError("write your pallas kernel here")


# speedup vs baseline: 53.0663x; 53.0663x over previous
import functools
import math

import jax
import jax.numpy as jnp
from jax import lax
from jax.experimental import pallas as pl
from jax.experimental.pallas import tpu as pltpu
from jax.experimental.pallas import tpu_sc as plsc

_B = 10000
_E = 320000
_IN = 1489
_NM, _NP, _NG = 167, 881, 441
_PM, _PP, _PG = 256, 896, 512
_SCALE = 1.0 / math.sqrt(64.0)

_NC, _NS = 2, 16
_NW = _NC * _NS
_CH = 16000

_SC_PLAN = {
    "m": (313, 1, _PM, _NM),
    "p": (105, 3, _PP, _NP),
    "g": (157, 2, _PG, _NG),
}


_NB = 2


def _sc_hist_body(fm, fp, fg, cm, cp, cg, hist, fb0, fb1, sem0, sem1):
    cid = lax.axis_index("c")
    sid = lax.axis_index("s")
    wid = sid * _NC + cid
    ones = jnp.ones((16,), jnp.float32)
    bufs = (fb0, fb1)
    sems = (sem0, sem1)
    nchunk = _E // _CH

    for key, f_hbm, c_hbm in (("m", fm, cm), ("p", fp, cp), ("g", fg, cg)):
        rows, npass, npad, _n = _SC_PLAN[key]
        nwords = rows * npad
        for p in range(npass):
            slot = p * _NW + wid
            base = slot * rows * npad

            for b in range(_NB):
                pltpu.async_copy(f_hbm.at[pl.ds(b * _CH, _CH)], bufs[b],
                                 sems[b])

            @plsc.parallel_loop(0, nwords // 16, unroll=8)
            def _zero(i):
                hist[pl.ds(i * 16, 16)] = jnp.zeros((16,), jnp.float32)

            @pl.loop(0, nchunk, step=_NB)
            def _chunk(ci):
                for b in range(_NB):
                    pltpu.make_async_copy(f_hbm.at[pl.ds(0, _CH)], bufs[b],
                                          sems[b]).wait()
                    buf = bufs[b]

                    @plsc.parallel_loop(0, _CH // 16, unroll=8)
                    def _vec(j):
                        f16 = buf[pl.ds(j * 16, 16)]
                        rel = f16 - base
                        msk = rel.astype(jnp.uint32) < jnp.uint32(nwords)
                        idx = jnp.where(msk, rel, 0)
                        plsc.addupdate_scatter(hist, [idx], ones, mask=msk)

                    nxt = ci + b + _NB

                    @pl.when(nxt < nchunk)
                    def _():
                        pltpu.async_copy(f_hbm.at[pl.ds(nxt * _CH, _CH)],
                                         bufs[b], sems[b])

            pltpu.sync_copy(hist.at[pl.ds(0, nwords)],
                            c_hbm.at[pl.ds(base, nwords)])


def _sc_hist(fm, fp, fg):
    mesh = plsc.VectorSubcoreMesh(core_axis_name="c", subcore_axis_name="s")
    out_type = (
        jax.ShapeDtypeStruct((_SC_PLAN["m"][0] * _NW * 1 * _PM,), jnp.float32),
        jax.ShapeDtypeStruct((_SC_PLAN["p"][0] * _NW * 3 * _PP,), jnp.float32),
        jax.ShapeDtypeStruct((_SC_PLAN["g"][0] * _NW * 2 * _PG,), jnp.float32),
    )
    fn = pl.kernel(
        _sc_hist_body,
        out_type=out_type,
        mesh=mesh,
        compiler_params=pltpu.CompilerParams(needs_layout_passes=False),
        scratch_types=[
            pltpu.VMEM((105 * 896,), jnp.float32),
            pltpu.VMEM((_CH,), jnp.int32),
            pltpu.VMEM((_CH,), jnp.int32),
            pltpu.SemaphoreType.DMA,
            pltpu.SemaphoreType.DMA,
        ],
    )
    return fn(fm, fp, fg)


_BR = 400


def _tcq_body(he, w1, w2, bc, out, wc):
    @pl.when(pl.program_id(0) == 0)
    def _():
        wc[...] = lax.dot_general(w1[...], w2[...], (((0,), (1,)), ((), ())),
                                  preferred_element_type=jnp.float32)

    out[...] = lax.dot_general(he[...], wc[...], (((1,), (0,)), ((), ())),
                               preferred_element_type=jnp.float32) + bc[...]


def _tcq(he, w1, w2, bc, interpret=False):
    grid = _B // _BR
    full = lambda shape: pl.BlockSpec(shape, lambda i: (0, 0))
    return pl.pallas_call(
        _tcq_body,
        grid=(grid,),
        in_specs=[
            pl.BlockSpec((_BR, _IN), lambda i: (i, 0)),
            full(w1.shape), full(w2.shape), full(bc.shape),
        ],
        out_specs=pl.BlockSpec((_BR, 64), lambda i: (i, 0)),
        out_shape=jax.ShapeDtypeStruct((_B, 64), jnp.float32),
        scratch_shapes=[pltpu.VMEM((_IN, 64), jnp.float32)],
        interpret=interpret,
    )(he, w1, w2, bc)


def _tc_body(q_ref, cm, cp, cg, k2, v2e, wm1, bm1, wm2, bm2, out):
    q = q_ref[...]
    s = lax.dot_general(q, k2[...], (((1,), (1,)), ((), ())),
                        preferred_element_type=jnp.float32) * _SCALE
    a = jnp.maximum(s, 0.01 * s)
    e = jnp.exp(a)

    msgs = []
    for cref, k0, k1 in ((cm, 0, _PM), (cp, _PM, _PM + _PP),
                         (cg, _PM + _PP, _PM + _PP + _PG)):
        wgt = cref[...] * e[:, k0:k1]
        me = lax.dot_general(wgt, v2e[pl.ds(k0, k1 - k0), :],
                             (((1,), (0,)), ((), ())),
                             preferred_element_type=jnp.float32)
        den = me[:, 64:65]
        den = jnp.where(den == 0.0, 1.0, den)
        msgs.append(me[:, :64] / den)

    he_new = jnp.concatenate(msgs, axis=1)
    h1 = lax.dot_general(he_new, wm1[...], (((1,), (1,)), ((), ())),
                         preferred_element_type=jnp.float32) + bm1[...]
    h1 = jnp.maximum(h1, 0.0)
    h2 = lax.dot_general(h1, wm2[...], (((1,), (1,)), ((), ())),
                         preferred_element_type=jnp.float32) + bm2[...]
    out[...] = jnp.maximum(h2, 0.0)


def _tc_fused(q, cm, cp, cg, k2, v2e, wm1, bm1, wm2, bm2, interpret=False):
    grid = _B // _BR
    full = lambda shape: pl.BlockSpec(shape, lambda i: (0, 0))
    return pl.pallas_call(
        _tc_body,
        grid=(grid,),
        in_specs=[
            pl.BlockSpec((_BR, 64), lambda i: (i, 0)),
            pl.BlockSpec((_BR, _PM), lambda i: (i, 0)),
            pl.BlockSpec((_BR, _PP), lambda i: (i, 0)),
            pl.BlockSpec((_BR, _PG), lambda i: (i, 0)),
            full(k2.shape), full(v2e.shape),
            full(wm1.shape), full(bm1.shape), full(wm2.shape), full(bm2.shape),
        ],
        out_specs=pl.BlockSpec((_BR, 64), lambda i: (i, 0)),
        out_shape=jax.ShapeDtypeStruct((_B, 64), jnp.float32),
        interpret=interpret,
    )(q, cm, cp, cg, k2, v2e, wm1, bm1, wm2, bm2)


def kernel(he_feat, maccs_feat, pubchem_feat, erg_feat, src_maccs, dst_maccs,
           src_pubchem, dst_pubchem, src_erg, dst_erg, W1, b1, W2, b2, W3, b3,
           W4, b4, W5m, b5m, W5p, b5p, W5e, b5e, W6m, b6m, W7m, b7m, W6p, b6p,
           W7p, b7p, W6e, b6e, W7e, b7e, Wm1, bm1, Wm2, bm2):
    f32 = jnp.float32
    i32 = jnp.int32

    bc = (b1 @ W2.T + b2).reshape(1, 64).astype(f32)
    k2s, v2s = [], []
    for feat, W5, b5, W6, b6, W7, b7, npad in (
        (maccs_feat, W5m, b5m, W6m, b6m, W7m, b7m, _PM),
        (pubchem_feat, W5p, b5p, W6p, b6p, W7p, b7p, _PP),
        (erg_feat, W5e, b5e, W6e, b6e, W7e, b7e, _PG),
    ):
        qn = feat @ W5.T + b5
        k2 = qn @ W6.T + b6
        v2 = qn @ W7.T + b7
        pad = npad - k2.shape[0]
        k2s.append(jnp.pad(k2, ((0, pad), (0, 0))))
        v2s.append(jnp.pad(v2, ((0, pad), (0, 0))))
    k2 = jnp.concatenate(k2s, axis=0).astype(f32)
    v2 = jnp.concatenate(v2s, axis=0).astype(f32)
    npad_tot = _PM + _PP + _PG
    v2e = jnp.zeros((npad_tot, 128), f32)
    v2e = v2e.at[:, :64].set(v2).at[:, 64].set(1.0)

    fm = dst_maccs.astype(i32) * _PM + src_maccs.astype(i32)
    fp = dst_pubchem.astype(i32) * _PP + src_pubchem.astype(i32)
    fg = dst_erg.astype(i32) * _PG + src_erg.astype(i32)
    cm, cp, cg = _sc_hist(fm, fp, fg)
    cm = cm.reshape(-1, _PM)
    cp = cp.reshape(-1, _PP)
    cg = cg.reshape(-1, _PG)

    q = _tcq(he_feat.astype(f32), W1.astype(f32), W2.astype(f32), bc)
    return _tc_fused(q, cm, cp, cg, k2, v2e, Wm1.astype(f32),
                     bm1.reshape(1, -1).astype(f32), Wm2.astype(f32),
                     bm2.reshape(1, -1).astype(f32))


# speedup vs baseline: 70.7313x; 1.3329x over previous
import functools
import math

import jax
import jax.numpy as jnp
from jax import lax
from jax.experimental import pallas as pl
from jax.experimental.pallas import tpu as pltpu
from jax.experimental.pallas import tpu_sc as plsc

_B = 10000
_E = 320000
_IN = 1489
_NM, _NP, _NG = 167, 881, 441
_PM, _PP, _PG = 256, 1024, 512
_SCALE = 1.0 / math.sqrt(64.0)

_NC, _NS = 2, 16
_NW = _NC * _NS
_CH = 16000

_SC_PLAN = {
    "m": (313, 1, _PM, _NM),
    "p": (157, 2, _PP, _NP),
    "g": (313, 1, _PG, _NG),
}


_NB = 2


def _sc_hist_body(fm, fp, fg, cm, cp, cg, hist, fb0, fb1, sem0, sem1):
    cid = lax.axis_index("c")
    sid = lax.axis_index("s")
    wid = sid * _NC + cid
    ones = jnp.ones((16,), jnp.float32)
    bufs = (fb0, fb1)
    sems = (sem0, sem1)
    nchunk = _E // _CH

    for key, f_hbm, c_hbm in (("m", fm, cm), ("p", fp, cp), ("g", fg, cg)):
        rows, npass, npad, _n = _SC_PLAN[key]
        span = rows * npad
        nwords = span // 2
        for p in range(npass):
            slot = p * _NW + wid
            base = slot * span

            for b in range(_NB):
                pltpu.async_copy(f_hbm.at[pl.ds(b * _CH, _CH)], bufs[b],
                                 sems[b])

            @plsc.parallel_loop(0, nwords // 16, unroll=8)
            def _zero(i):
                hist[pl.ds(i * 16, 16)] = jnp.zeros((16,), jnp.int32)

            @pl.loop(0, nchunk, step=_NB)
            def _chunk(ci):
                for b in range(_NB):
                    pltpu.make_async_copy(f_hbm.at[pl.ds(0, _CH)], bufs[b],
                                          sems[b]).wait()
                    buf = bufs[b]

                    @plsc.parallel_loop(0, _CH // 16, unroll=8)
                    def _vec(j):
                        f16 = buf[pl.ds(j * 16, 16)]
                        rel = f16 - base
                        msk = rel.astype(jnp.uint32) < jnp.uint32(span)
                        idx = jnp.where(msk, rel >> 1, 0)
                        inc = jnp.where((rel & 1) == 1,
                                        jnp.int32(65536), jnp.int32(1))
                        plsc.addupdate_scatter(hist, [idx], inc, mask=msk)

                    nxt = ci + b + _NB

                    @pl.when(nxt < nchunk)
                    def _():
                        pltpu.async_copy(f_hbm.at[pl.ds(nxt * _CH, _CH)],
                                         bufs[b], sems[b])

            pltpu.sync_copy(hist.at[pl.ds(0, nwords)],
                            c_hbm.at[pl.ds(slot * nwords, nwords)])


def _sc_hist(fm, fp, fg):
    mesh = plsc.VectorSubcoreMesh(core_axis_name="c", subcore_axis_name="s")
    out_type = (
        jax.ShapeDtypeStruct((_SC_PLAN["m"][0] * _NW * 1 * _PM // 2,),
                             jnp.int32),
        jax.ShapeDtypeStruct((_SC_PLAN["p"][0] * _NW * 2 * _PP // 2,),
                             jnp.int32),
        jax.ShapeDtypeStruct((_SC_PLAN["g"][0] * _NW * 1 * _PG // 2,),
                             jnp.int32),
    )
    fn = pl.kernel(
        _sc_hist_body,
        out_type=out_type,
        mesh=mesh,
        compiler_params=pltpu.CompilerParams(needs_layout_passes=False),
        scratch_types=[
            pltpu.VMEM((157 * 512,), jnp.int32),
            pltpu.VMEM((_CH,), jnp.int32),
            pltpu.VMEM((_CH,), jnp.int32),
            pltpu.SemaphoreType.DMA,
            pltpu.SemaphoreType.DMA,
        ],
    )
    return fn(fm, fp, fg)


_BR = 400


def _tcq_body(he, w1, w2, bc, out, wc):
    @pl.when(pl.program_id(0) == 0)
    def _():
        wc[...] = lax.dot_general(w1[...], w2[...], (((0,), (1,)), ((), ())),
                                  preferred_element_type=jnp.float32)

    out[...] = lax.dot_general(he[...], wc[...], (((1,), (0,)), ((), ())),
                               preferred_element_type=jnp.float32) + bc[...]


def _tcq(he, w1, w2, bc, interpret=False):
    grid = _B // _BR
    full = lambda shape: pl.BlockSpec(shape, lambda i: (0, 0))
    return pl.pallas_call(
        _tcq_body,
        grid=(grid,),
        in_specs=[
            pl.BlockSpec((_BR, _IN), lambda i: (i, 0)),
            full(w1.shape), full(w2.shape), full(bc.shape),
        ],
        out_specs=pl.BlockSpec((_BR, 64), lambda i: (i, 0)),
        out_shape=jax.ShapeDtypeStruct((_B, 64), jnp.float32),
        scratch_shapes=[pltpu.VMEM((_IN, 64), jnp.float32)],
        interpret=interpret,
    )(he, w1, w2, bc)


def _tc_body(q_ref, cm, cp, cg, k2, v2e, wm1, bm1, wm2, bm2, out):
    q = q_ref[...]
    s = lax.dot_general(q, k2[...], (((1,), (1,)), ((), ())),
                        preferred_element_type=jnp.float32) * _SCALE
    a = jnp.maximum(s, 0.01 * s)
    e = jnp.exp(a)

    msgs = []
    for cref, k0, k1 in ((cm, 0, _PM), (cp, _PM, _PM + _PP),
                         (cg, _PM + _PP, _PM + _PP + _PG)):
        cpk = cref[...]
        clo = (cpk & 0xFFFF).astype(jnp.float32)
        chi = lax.shift_right_logical(cpk, 16).astype(jnp.float32)
        c = jnp.concatenate([clo, chi], axis=1)
        wgt = c * e[:, k0:k1]
        me = lax.dot_general(wgt, v2e[pl.ds(k0, k1 - k0), :],
                             (((1,), (0,)), ((), ())),
                             preferred_element_type=jnp.float32)
        den = me[:, 64:65]
        den = jnp.where(den == 0.0, 1.0, den)
        msgs.append(me[:, :64] / den)

    he_new = jnp.concatenate(msgs, axis=1)
    h1 = lax.dot_general(he_new, wm1[...], (((1,), (1,)), ((), ())),
                         preferred_element_type=jnp.float32) + bm1[...]
    h1 = jnp.maximum(h1, 0.0)
    h2 = lax.dot_general(h1, wm2[...], (((1,), (1,)), ((), ())),
                         preferred_element_type=jnp.float32) + bm2[...]
    out[...] = jnp.maximum(h2, 0.0)


def _tc_fused(q, cm, cp, cg, k2, v2e, wm1, bm1, wm2, bm2, interpret=False):
    grid = _B // _BR
    full = lambda shape: pl.BlockSpec(shape, lambda i: (0, 0))
    return pl.pallas_call(
        _tc_body,
        grid=(grid,),
        in_specs=[
            pl.BlockSpec((_BR, 64), lambda i: (i, 0)),
            pl.BlockSpec((_BR, _PM // 2), lambda i: (i, 0)),
            pl.BlockSpec((_BR, _PP // 2), lambda i: (i, 0)),
            pl.BlockSpec((_BR, _PG // 2), lambda i: (i, 0)),
            full(k2.shape), full(v2e.shape),
            full(wm1.shape), full(bm1.shape), full(wm2.shape), full(bm2.shape),
        ],
        out_specs=pl.BlockSpec((_BR, 64), lambda i: (i, 0)),
        out_shape=jax.ShapeDtypeStruct((_B, 64), jnp.float32),
        interpret=interpret,
    )(q, cm, cp, cg, k2, v2e, wm1, bm1, wm2, bm2)


def kernel(he_feat, maccs_feat, pubchem_feat, erg_feat, src_maccs, dst_maccs,
           src_pubchem, dst_pubchem, src_erg, dst_erg, W1, b1, W2, b2, W3, b3,
           W4, b4, W5m, b5m, W5p, b5p, W5e, b5e, W6m, b6m, W7m, b7m, W6p, b6p,
           W7p, b7p, W6e, b6e, W7e, b7e, Wm1, bm1, Wm2, bm2):
    f32 = jnp.float32
    i32 = jnp.int32

    bc = (b1 @ W2.T + b2).reshape(1, 64).astype(f32)
    k2s, v2s = [], []
    for feat, W5, b5, W6, b6, W7, b7, npad in (
        (maccs_feat, W5m, b5m, W6m, b6m, W7m, b7m, _PM),
        (pubchem_feat, W5p, b5p, W6p, b6p, W7p, b7p, _PP),
        (erg_feat, W5e, b5e, W6e, b6e, W7e, b7e, _PG),
    ):
        qn = feat @ W5.T + b5
        k2 = qn @ W6.T + b6
        v2 = qn @ W7.T + b7
        pad = npad - k2.shape[0]
        k2s.append(jnp.pad(k2, ((0, pad), (0, 0))))
        v2s.append(jnp.pad(v2, ((0, pad), (0, 0))))
    k2 = jnp.concatenate(k2s, axis=0).astype(f32)
    v2 = jnp.concatenate(v2s, axis=0).astype(f32)
    npad_tot = _PM + _PP + _PG
    v2e = jnp.zeros((npad_tot, 128), f32)
    v2e = v2e.at[:, :64].set(v2).at[:, 64].set(1.0)

    def _flat(d, s, npad):
        d = d.astype(i32)
        s = s.astype(i32)
        h = npad // 2
        return d * npad + (s & (h - 1)) * 2 + (s >= h).astype(i32)

    fm = _flat(dst_maccs, src_maccs, _PM)
    fp = _flat(dst_pubchem, src_pubchem, _PP)
    fg = _flat(dst_erg, src_erg, _PG)
    cm, cp, cg = _sc_hist(fm, fp, fg)
    cm = cm.reshape(-1, _PM // 2)
    cp = cp.reshape(-1, _PP // 2)
    cg = cg.reshape(-1, _PG // 2)

    q = _tcq(he_feat.astype(f32), W1.astype(f32), W2.astype(f32), bc)
    return _tc_fused(q, cm, cp, cg, k2, v2e, Wm1.astype(f32),
                     bm1.reshape(1, -1).astype(f32), Wm2.astype(f32),
                     bm2.reshape(1, -1).astype(f32))


# speedup vs baseline: 71.7799x; 1.0148x over previous
import functools
import math

import jax
import jax.numpy as jnp
from jax import lax
from jax.experimental import pallas as pl
from jax.experimental.pallas import tpu as pltpu
from jax.experimental.pallas import tpu_sc as plsc

_B = 10000
_E = 320000
_IN = 1489
_NM, _NP, _NG = 167, 881, 441
_PM, _PP, _PG = 256, 1024, 512
_SCALE = 1.0 / math.sqrt(64.0)

_NC, _NS = 2, 16
_NW = _NC * _NS
_CH = 16000

_SC_PLAN = {
    "m": (313, 1, _PM, _NM),
    "p": (157, 2, _PP, _NP),
    "g": (313, 1, _PG, _NG),
}


_NB = 2


def _sc_hist_body(fm, fp, fg, cm, cp, cg, hist, fb0, fb1, sem0, sem1):
    cid = lax.axis_index("c")
    sid = lax.axis_index("s")
    wid = sid * _NC + cid
    ones = jnp.ones((16,), jnp.float32)
    bufs = (fb0, fb1)
    sems = (sem0, sem1)
    nchunk = _E // _CH

    for key, f_hbm, c_hbm in (("m", fm, cm), ("p", fp, cp), ("g", fg, cg)):
        rows, npass, npad, _n = _SC_PLAN[key]
        span = rows * npad
        nwords = span // 2
        for p in range(npass):
            slot = p * _NW + wid
            base = slot * span

            for b in range(_NB):
                pltpu.async_copy(f_hbm.at[pl.ds(b * _CH, _CH)], bufs[b],
                                 sems[b])

            @plsc.parallel_loop(0, nwords // 16, unroll=8)
            def _zero(i):
                hist[pl.ds(i * 16, 16)] = jnp.zeros((16,), jnp.int32)

            @pl.loop(0, nchunk, step=_NB)
            def _chunk(ci):
                for b in range(_NB):
                    pltpu.make_async_copy(f_hbm.at[pl.ds(0, _CH)], bufs[b],
                                          sems[b]).wait()
                    buf = bufs[b]

                    @plsc.parallel_loop(0, _CH // 16, unroll=8)
                    def _vec(j):
                        f16 = buf[pl.ds(j * 16, 16)]
                        rel = f16 - base
                        msk = rel.astype(jnp.uint32) < jnp.uint32(span)
                        idx = rel >> 1
                        inc = jnp.int32(1) << ((rel & 1) << 4)
                        plsc.addupdate_scatter(hist, [idx], inc, mask=msk)

                    nxt = ci + b + _NB

                    @pl.when(nxt < nchunk)
                    def _():
                        pltpu.async_copy(f_hbm.at[pl.ds(nxt * _CH, _CH)],
                                         bufs[b], sems[b])

            pltpu.sync_copy(hist.at[pl.ds(0, nwords)],
                            c_hbm.at[pl.ds(slot * nwords, nwords)])


def _sc_hist(fm, fp, fg):
    mesh = plsc.VectorSubcoreMesh(core_axis_name="c", subcore_axis_name="s")
    out_type = (
        jax.ShapeDtypeStruct((_SC_PLAN["m"][0] * _NW * 1 * _PM // 2,),
                             jnp.int32),
        jax.ShapeDtypeStruct((_SC_PLAN["p"][0] * _NW * 2 * _PP // 2,),
                             jnp.int32),
        jax.ShapeDtypeStruct((_SC_PLAN["g"][0] * _NW * 1 * _PG // 2,),
                             jnp.int32),
    )
    fn = pl.kernel(
        _sc_hist_body,
        out_type=out_type,
        mesh=mesh,
        compiler_params=pltpu.CompilerParams(needs_layout_passes=False),
        scratch_types=[
            pltpu.VMEM((157 * 512,), jnp.int32),
            pltpu.VMEM((_CH,), jnp.int32),
            pltpu.VMEM((_CH,), jnp.int32),
            pltpu.SemaphoreType.DMA,
            pltpu.SemaphoreType.DMA,
        ],
    )
    return fn(fm, fp, fg)


_BR = 400


def _tcq_body(he, w1, w2, bc, out, wc):
    @pl.when(pl.program_id(0) == 0)
    def _():
        wc[...] = lax.dot_general(w1[...], w2[...], (((0,), (1,)), ((), ())),
                                  preferred_element_type=jnp.float32)

    out[...] = lax.dot_general(he[...], wc[...], (((1,), (0,)), ((), ())),
                               preferred_element_type=jnp.float32) + bc[...]


def _tcq(he, w1, w2, bc, interpret=False):
    grid = _B // _BR
    full = lambda shape: pl.BlockSpec(shape, lambda i: (0, 0))
    return pl.pallas_call(
        _tcq_body,
        grid=(grid,),
        in_specs=[
            pl.BlockSpec((_BR, _IN), lambda i: (i, 0)),
            full(w1.shape), full(w2.shape), full(bc.shape),
        ],
        out_specs=pl.BlockSpec((_BR, 64), lambda i: (i, 0)),
        out_shape=jax.ShapeDtypeStruct((_B, 64), jnp.float32),
        scratch_shapes=[pltpu.VMEM((_IN, 64), jnp.float32)],
        interpret=interpret,
    )(he, w1, w2, bc)


def _tc_body(q_ref, cm, cp, cg, k2, v2e, wm1, bm1, wm2, bm2, out):
    q = q_ref[...]
    s = lax.dot_general(q, k2[...], (((1,), (1,)), ((), ())),
                        preferred_element_type=jnp.float32) * _SCALE
    a = jnp.maximum(s, 0.01 * s)
    e = jnp.exp(a)

    msgs = []
    for cref, k0, k1 in ((cm, 0, _PM), (cp, _PM, _PM + _PP),
                         (cg, _PM + _PP, _PM + _PP + _PG)):
        cpk = cref[...]
        clo = (cpk & 0xFFFF).astype(jnp.float32)
        chi = lax.shift_right_logical(cpk, 16).astype(jnp.float32)
        c = jnp.concatenate([clo, chi], axis=1)
        wgt = c * e[:, k0:k1]
        me = lax.dot_general(wgt, v2e[pl.ds(k0, k1 - k0), :],
                             (((1,), (0,)), ((), ())),
                             preferred_element_type=jnp.float32)
        den = me[:, 64:65]
        den = jnp.where(den == 0.0, 1.0, den)
        msgs.append(me[:, :64] / den)

    he_new = jnp.concatenate(msgs, axis=1)
    h1 = lax.dot_general(he_new, wm1[...], (((1,), (1,)), ((), ())),
                         preferred_element_type=jnp.float32) + bm1[...]
    h1 = jnp.maximum(h1, 0.0)
    h2 = lax.dot_general(h1, wm2[...], (((1,), (1,)), ((), ())),
                         preferred_element_type=jnp.float32) + bm2[...]
    out[...] = jnp.maximum(h2, 0.0)


def _tc_fused(q, cm, cp, cg, k2, v2e, wm1, bm1, wm2, bm2, interpret=False):
    grid = _B // _BR
    full = lambda shape: pl.BlockSpec(shape, lambda i: (0, 0))
    return pl.pallas_call(
        _tc_body,
        grid=(grid,),
        in_specs=[
            pl.BlockSpec((_BR, 64), lambda i: (i, 0)),
            pl.BlockSpec((_BR, _PM // 2), lambda i: (i, 0)),
            pl.BlockSpec((_BR, _PP // 2), lambda i: (i, 0)),
            pl.BlockSpec((_BR, _PG // 2), lambda i: (i, 0)),
            full(k2.shape), full(v2e.shape),
            full(wm1.shape), full(bm1.shape), full(wm2.shape), full(bm2.shape),
        ],
        out_specs=pl.BlockSpec((_BR, 64), lambda i: (i, 0)),
        out_shape=jax.ShapeDtypeStruct((_B, 64), jnp.float32),
        interpret=interpret,
    )(q, cm, cp, cg, k2, v2e, wm1, bm1, wm2, bm2)


def kernel(he_feat, maccs_feat, pubchem_feat, erg_feat, src_maccs, dst_maccs,
           src_pubchem, dst_pubchem, src_erg, dst_erg, W1, b1, W2, b2, W3, b3,
           W4, b4, W5m, b5m, W5p, b5p, W5e, b5e, W6m, b6m, W7m, b7m, W6p, b6p,
           W7p, b7p, W6e, b6e, W7e, b7e, Wm1, bm1, Wm2, bm2):
    f32 = jnp.float32
    i32 = jnp.int32

    bc = (b1 @ W2.T + b2).reshape(1, 64).astype(f32)
    k2s, v2s = [], []
    for feat, W5, b5, W6, b6, W7, b7, npad in (
        (maccs_feat, W5m, b5m, W6m, b6m, W7m, b7m, _PM),
        (pubchem_feat, W5p, b5p, W6p, b6p, W7p, b7p, _PP),
        (erg_feat, W5e, b5e, W6e, b6e, W7e, b7e, _PG),
    ):
        qn = feat @ W5.T + b5
        k2 = qn @ W6.T + b6
        v2 = qn @ W7.T + b7
        pad = npad - k2.shape[0]
        k2s.append(jnp.pad(k2, ((0, pad), (0, 0))))
        v2s.append(jnp.pad(v2, ((0, pad), (0, 0))))
    k2 = jnp.concatenate(k2s, axis=0).astype(f32)
    v2 = jnp.concatenate(v2s, axis=0).astype(f32)
    npad_tot = _PM + _PP + _PG
    v2e = jnp.zeros((npad_tot, 128), f32)
    v2e = v2e.at[:, :64].set(v2).at[:, 64].set(1.0)

    def _flat(d, s, npad):
        d = d.astype(i32)
        s = s.astype(i32)
        h = npad // 2
        return d * npad + (s & (h - 1)) * 2 + (s >= h).astype(i32)

    fm = _flat(dst_maccs, src_maccs, _PM)
    fp = _flat(dst_pubchem, src_pubchem, _PP)
    fg = _flat(dst_erg, src_erg, _PG)
    cm, cp, cg = _sc_hist(fm, fp, fg)
    cm = cm.reshape(-1, _PM // 2)
    cp = cp.reshape(-1, _PP // 2)
    cg = cg.reshape(-1, _PG // 2)

    q = _tcq(he_feat.astype(f32), W1.astype(f32), W2.astype(f32), bc)
    return _tc_fused(q, cm, cp, cg, k2, v2e, Wm1.astype(f32),
                     bm1.reshape(1, -1).astype(f32), Wm2.astype(f32),
                     bm2.reshape(1, -1).astype(f32))


# speedup vs baseline: 74.1006x; 1.0323x over previous
import functools
import math

import jax
import jax.numpy as jnp
from jax import lax
from jax.experimental import pallas as pl
from jax.experimental.pallas import tpu as pltpu
from jax.experimental.pallas import tpu_sc as plsc

_B = 10000
_E = 320000
_IN = 1489
_NM, _NP, _NG = 167, 881, 441
_PM, _PP, _PG = 256, 1024, 512
_SCALE = 1.0 / math.sqrt(64.0)

_NC, _NS = 2, 16
_NW = _NC * _NS
_CH = 16000

_SC_PLAN = {
    "m": (313, 1, _PM, _NM),
    "p": (157, 2, _PP, _NP),
    "g": (313, 1, _PG, _NG),
}


_NB = 2


def _sc_hist_body(fm, fp, fg, cm, cp, cg, hist, fb0, fb1, sem0, sem1):
    cid = lax.axis_index("c")
    sid = lax.axis_index("s")
    wid = sid * _NC + cid
    ones = jnp.ones((16,), jnp.float32)
    bufs = (fb0, fb1)
    sems = (sem0, sem1)
    nchunk = _E // _CH

    for key, f_hbm, c_hbm in (("m", fm, cm), ("p", fp, cp), ("g", fg, cg)):
        rows, npass, npad, _n = _SC_PLAN[key]
        span = rows * npad
        nwords = span // 2
        for p in range(npass):
            slot = p * _NW + wid
            base = slot * span

            for b in range(_NB):
                pltpu.async_copy(f_hbm.at[pl.ds(b * _CH, _CH)], bufs[b],
                                 sems[b])

            @plsc.parallel_loop(0, nwords // 16, unroll=8)
            def _zero(i):
                hist[pl.ds(i * 16, 16)] = jnp.zeros((16,), jnp.int32)

            @pl.loop(0, nchunk, step=_NB)
            def _chunk(ci):
                for b in range(_NB):
                    pltpu.make_async_copy(f_hbm.at[pl.ds(0, _CH)], bufs[b],
                                          sems[b]).wait()
                    buf = bufs[b]

                    @plsc.parallel_loop(0, _CH // 16, unroll=8)
                    def _vec(j):
                        f16 = buf[pl.ds(j * 16, 16)]
                        rel = f16 - base
                        msk = rel.astype(jnp.uint32) < jnp.uint32(span)
                        idx = rel >> 1
                        inc = jnp.int32(1) << ((rel & 1) << 4)
                        plsc.addupdate_scatter(hist, [idx], inc, mask=msk)

                    nxt = ci + b + _NB

                    @pl.when(nxt < nchunk)
                    def _():
                        pltpu.async_copy(f_hbm.at[pl.ds(nxt * _CH, _CH)],
                                         bufs[b], sems[b])

            pltpu.sync_copy(hist.at[pl.ds(0, nwords)],
                            c_hbm.at[pl.ds(slot * nwords, nwords)])


def _sc_hist(fm, fp, fg):
    mesh = plsc.VectorSubcoreMesh(core_axis_name="c", subcore_axis_name="s")
    out_type = (
        jax.ShapeDtypeStruct((_SC_PLAN["m"][0] * _NW * 1 * _PM // 2,),
                             jnp.int32),
        jax.ShapeDtypeStruct((_SC_PLAN["p"][0] * _NW * 2 * _PP // 2,),
                             jnp.int32),
        jax.ShapeDtypeStruct((_SC_PLAN["g"][0] * _NW * 1 * _PG // 2,),
                             jnp.int32),
    )
    fn = pl.kernel(
        _sc_hist_body,
        out_type=out_type,
        mesh=mesh,
        compiler_params=pltpu.CompilerParams(needs_layout_passes=False),
        scratch_types=[
            pltpu.VMEM((157 * 512,), jnp.int32),
            pltpu.VMEM((_CH,), jnp.int32),
            pltpu.VMEM((_CH,), jnp.int32),
            pltpu.SemaphoreType.DMA,
            pltpu.SemaphoreType.DMA,
        ],
    )
    return fn(fm, fp, fg)


_BR = 1000


def _tcq_body(he, w1, w2, bc, out, wc):
    @pl.when(pl.program_id(0) == 0)
    def _():
        wc[...] = lax.dot_general(w1[...], w2[...], (((0,), (1,)), ((), ())),
                                  preferred_element_type=jnp.float32)

    out[...] = lax.dot_general(he[...], wc[...], (((1,), (0,)), ((), ())),
                               preferred_element_type=jnp.float32) + bc[...]


def _tcq(he, w1, w2, bc, interpret=False):
    grid = _B // _BR
    full = lambda shape: pl.BlockSpec(shape, lambda i: (0, 0))
    return pl.pallas_call(
        _tcq_body,
        grid=(grid,),
        in_specs=[
            pl.BlockSpec((_BR, _IN), lambda i: (i, 0)),
            full(w1.shape), full(w2.shape), full(bc.shape),
        ],
        out_specs=pl.BlockSpec((_BR, 64), lambda i: (i, 0)),
        out_shape=jax.ShapeDtypeStruct((_B, 64), jnp.float32),
        scratch_shapes=[pltpu.VMEM((_IN, 64), jnp.float32)],
        interpret=interpret,
    )(he, w1, w2, bc)


def _tc_body(q_ref, cm, cp, cg, k2, v2e, wm1, bm1, wm2, bm2, out):
    q = q_ref[...]
    s = lax.dot_general(q, k2[...], (((1,), (1,)), ((), ())),
                        preferred_element_type=jnp.float32) * _SCALE
    a = jnp.maximum(s, 0.01 * s)
    e = jnp.exp(a)

    msgs = []
    for cref, k0, k1 in ((cm, 0, _PM), (cp, _PM, _PM + _PP),
                         (cg, _PM + _PP, _PM + _PP + _PG)):
        cpk = cref[...]
        clo = (cpk & 0xFFFF).astype(jnp.float32)
        chi = lax.shift_right_logical(cpk, 16).astype(jnp.float32)
        c = jnp.concatenate([clo, chi], axis=1)
        wgt = c * e[:, k0:k1]
        me = lax.dot_general(wgt, v2e[pl.ds(k0, k1 - k0), :],
                             (((1,), (0,)), ((), ())),
                             preferred_element_type=jnp.float32)
        den = me[:, 64:65]
        den = jnp.where(den == 0.0, 1.0, den)
        msgs.append(me[:, :64] / den)

    he_new = jnp.concatenate(msgs, axis=1)
    h1 = lax.dot_general(he_new, wm1[...], (((1,), (1,)), ((), ())),
                         preferred_element_type=jnp.float32) + bm1[...]
    h1 = jnp.maximum(h1, 0.0)
    h2 = lax.dot_general(h1, wm2[...], (((1,), (1,)), ((), ())),
                         preferred_element_type=jnp.float32) + bm2[...]
    out[...] = jnp.maximum(h2, 0.0)


def _tc_fused(q, cm, cp, cg, k2, v2e, wm1, bm1, wm2, bm2, interpret=False):
    grid = _B // _BR
    full = lambda shape: pl.BlockSpec(shape, lambda i: (0, 0))
    return pl.pallas_call(
        _tc_body,
        grid=(grid,),
        in_specs=[
            pl.BlockSpec((_BR, 64), lambda i: (i, 0)),
            pl.BlockSpec((_BR, _PM // 2), lambda i: (i, 0)),
            pl.BlockSpec((_BR, _PP // 2), lambda i: (i, 0)),
            pl.BlockSpec((_BR, _PG // 2), lambda i: (i, 0)),
            full(k2.shape), full(v2e.shape),
            full(wm1.shape), full(bm1.shape), full(wm2.shape), full(bm2.shape),
        ],
        out_specs=pl.BlockSpec((_BR, 64), lambda i: (i, 0)),
        out_shape=jax.ShapeDtypeStruct((_B, 64), jnp.float32),
        interpret=interpret,
    )(q, cm, cp, cg, k2, v2e, wm1, bm1, wm2, bm2)


def kernel(he_feat, maccs_feat, pubchem_feat, erg_feat, src_maccs, dst_maccs,
           src_pubchem, dst_pubchem, src_erg, dst_erg, W1, b1, W2, b2, W3, b3,
           W4, b4, W5m, b5m, W5p, b5p, W5e, b5e, W6m, b6m, W7m, b7m, W6p, b6p,
           W7p, b7p, W6e, b6e, W7e, b7e, Wm1, bm1, Wm2, bm2):
    f32 = jnp.float32
    i32 = jnp.int32

    bc = (b1 @ W2.T + b2).reshape(1, 64).astype(f32)
    k2s, v2s = [], []
    for feat, W5, b5, W6, b6, W7, b7, npad in (
        (maccs_feat, W5m, b5m, W6m, b6m, W7m, b7m, _PM),
        (pubchem_feat, W5p, b5p, W6p, b6p, W7p, b7p, _PP),
        (erg_feat, W5e, b5e, W6e, b6e, W7e, b7e, _PG),
    ):
        qn = feat @ W5.T + b5
        k2 = qn @ W6.T + b6
        v2 = qn @ W7.T + b7
        pad = npad - k2.shape[0]
        k2s.append(jnp.pad(k2, ((0, pad), (0, 0))))
        v2s.append(jnp.pad(v2, ((0, pad), (0, 0))))
    k2 = jnp.concatenate(k2s, axis=0).astype(f32)
    v2 = jnp.concatenate(v2s, axis=0).astype(f32)
    npad_tot = _PM + _PP + _PG
    v2e = jnp.zeros((npad_tot, 128), f32)
    v2e = v2e.at[:, :64].set(v2).at[:, 64].set(1.0)

    def _flat(d, s, npad):
        d = d.astype(i32)
        s = s.astype(i32)
        h = npad // 2
        return d * npad + (s & (h - 1)) * 2 + (s >= h).astype(i32)

    fm = _flat(dst_maccs, src_maccs, _PM)
    fp = _flat(dst_pubchem, src_pubchem, _PP)
    fg = _flat(dst_erg, src_erg, _PG)
    cm, cp, cg = _sc_hist(fm, fp, fg)
    cm = cm.reshape(-1, _PM // 2)
    cp = cp.reshape(-1, _PP // 2)
    cg = cg.reshape(-1, _PG // 2)

    q = _tcq(he_feat.astype(f32), W1.astype(f32), W2.astype(f32), bc)
    return _tc_fused(q, cm, cp, cg, k2, v2e, Wm1.astype(f32),
                     bm1.reshape(1, -1).astype(f32), Wm2.astype(f32),
                     bm2.reshape(1, -1).astype(f32))
